# Initial kernel scaffold; baseline (speedup 1.0000x reference)
import functools

import jax
import jax.numpy as jnp
from jax import lax
from jax.experimental import pallas as pl
from jax.experimental.pallas import tpu as pltpu
from jax.experimental.pallas import tpu_sc as plsc

N = 10000
E = 320000
H = 128
NA = 19
EW = 16
DEG_COL = 10

NC = 2
NS = 16
NW = NC * NS
EPW = E // NW
CHUNK = 80
NCHUNK = EPW // CHUNK
RPT = N // NS

BN = 1000


def _make_sc_scatter(with_pre: bool):
  mesh = plsc.VectorSubcoreMesh(
      core_axis_name="c", subcore_axis_name="s",
      num_cores=NC, num_subcores=NS)

  out_type = [jax.ShapeDtypeStruct((NC, N, H), jnp.float32)]
  scratch = [
      pltpu.VMEM((NCHUNK, CHUNK), jnp.int32),
      pltpu.VMEM((NCHUNK, CHUNK), jnp.int32),
      pltpu.VMEM((CHUNK, H), jnp.float32),
      pltpu.VMEM_SHARED((N, H), jnp.float32),
      pltpu.SemaphoreType.DMA,
  ]
  if with_pre:
    out_type.append(jax.ShapeDtypeStruct((NC, N, EW), jnp.float32))
    scratch += [
        pltpu.VMEM((CHUNK, EW), jnp.float32),
        pltpu.VMEM_SHARED((N, EW), jnp.float32),
    ]

  def body(*refs):
    if with_pre:
      (b_hbm, srcr, dstr, zeros_hbm, zeros16_hbm, epad_hbm,
       p_out, e_out, src_v, dst_v, rows_v, p_sh, sem, ebuf_v, e_sh) = refs
    else:
      (b_hbm, srcr, dstr, zeros_hbm,
       p_out, src_v, dst_v, rows_v, p_sh, sem) = refs

    c = lax.axis_index("c")
    s = lax.axis_index("s")
    w = s * NC + c
    r0 = pl.multiple_of(s * RPT, RPT)

    pltpu.sync_copy(srcr.at[w], src_v)
    pltpu.sync_copy(dstr.at[w], dst_v)
    pltpu.sync_copy(zeros_hbm.at[pl.ds(r0, RPT)], p_sh.at[pl.ds(r0, RPT)])
    if with_pre:
      pltpu.sync_copy(zeros16_hbm.at[pl.ds(r0, RPT)], e_sh.at[pl.ds(r0, RPT)])
    plsc.subcore_barrier()

    @pl.loop(0, NCHUNK)
    def _(i):
      pltpu.async_copy(b_hbm.at[src_v.at[i]], rows_v, sem).wait()
      pltpu.sync_copy(rows_v, p_sh.at[dst_v.at[i]], add=True)
      if with_pre:
        pltpu.sync_copy(epad_hbm.at[w, i], ebuf_v)
        pltpu.sync_copy(ebuf_v, e_sh.at[dst_v.at[i]], add=True)

    plsc.subcore_barrier()
    pltpu.sync_copy(p_sh.at[pl.ds(r0, RPT)], p_out.at[c, pl.ds(r0, RPT)])
    if with_pre:
      pltpu.sync_copy(e_sh.at[pl.ds(r0, RPT)], e_out.at[c, pl.ds(r0, RPT)])

  return pl.kernel(body, out_type=out_type, mesh=mesh, scratch_types=scratch,
                   name="sc_scatter_pre" if with_pre else "sc_scatter")


_sc_scatter_pre = _make_sc_scatter(True)
_sc_scatter = _make_sc_scatter(False)


def _dot_t(a, w):
  return lax.dot_general(a, w, (((1,), (1,)), ((), ())),
                         preferred_element_type=jnp.float32)


def _tc0_body(x_ref, wn_ref, us_ref, h_ref, b_ref):
  h = _dot_t(x_ref[...], wn_ref[...])
  h_ref[...] = h
  b_ref[...] = _dot_t(h, us_ref[...])


def _gru_block(h, p_ref, es_ref, ud_ref, w16_ref, wih_ref, whh_ref,
               bih_ref, bhh_ref):
  es = es_ref[0] + es_ref[1]
  deg = es[:, DEG_COL:DEG_COL + 1]
  init = deg * _dot_t(h, ud_ref[...]) + _dot_t(es, w16_ref[...])
  agg = init + p_ref[0] + p_ref[1]
  gi = _dot_t(agg, wih_ref[...]) + bih_ref[...]
  gh = _dot_t(h, whh_ref[...]) + bhh_ref[...]
  r = jax.nn.sigmoid(gi[:, :H] + gh[:, :H])
  z = jax.nn.sigmoid(gi[:, H:2 * H] + gh[:, H:2 * H])
  n = jnp.tanh(gi[:, 2 * H:] + r * gh[:, 2 * H:])
  return (1.0 - z) * n + z * h


def _tcr_body(h_ref, p_ref, es_ref, ud_ref, w16_ref, wih_ref, whh_ref,
              bih_ref, bhh_ref, usn_ref, h_out_ref, b_out_ref):
  hn = _gru_block(h_ref[...], p_ref, es_ref, ud_ref, w16_ref,
                  wih_ref, whh_ref, bih_ref, bhh_ref)
  h_out_ref[...] = hn
  b_out_ref[...] = _dot_t(hn, usn_ref[...])


def _tc3_body(h_ref, p_ref, es_ref, ud_ref, w16_ref, wih_ref, whh_ref,
              bih_ref, bhh_ref, gv_w, gv_b, gvc_w, gvc_b, ev_w, ev_b,
              evc_w, evc_b, mean_w, mean_b, logvar_w, logvar_b,
              out_ref, acc_gv, acc_enc):
  hn = _gru_block(h_ref[...], p_ref, es_ref, ud_ref, w16_ref,
                  wih_ref, whh_ref, bih_ref, bhh_ref)
  gvp = jnp.sum(jax.nn.sigmoid(_dot_t(hn, gvc_w[...]) + gvc_b[...])
                * (_dot_t(hn, gv_w[...]) + gv_b[...]), axis=0, keepdims=True)
  encp = jnp.sum(jax.nn.sigmoid(_dot_t(hn, evc_w[...]) + evc_b[...])
                 * (_dot_t(hn, ev_w[...]) + ev_b[...]), axis=0, keepdims=True)
  k = pl.program_id(0)

  @pl.when(k == 0)
  def _():
    acc_gv[...] = gvp
    acc_enc[...] = encp

  @pl.when(k > 0)
  def _():
    acc_gv[...] = acc_gv[...] + gvp
    acc_enc[...] = acc_enc[...] + encp

  @pl.when(k == pl.num_programs(0) - 1)
  def _():
    enc = acc_enc[...]
    mean = _dot_t(enc, mean_w[...]) + mean_b[...]
    logvar = _dot_t(enc, logvar_w[...]) + logvar_b[...]
    out_ref[...] = jnp.concatenate([mean, logvar, acc_gv[...]], axis=1)


def _full(shape):
  return pl.BlockSpec(shape, lambda i: (0,) * len(shape))


def _rows(shape):
  return pl.BlockSpec(shape, lambda i: (i,) + (0,) * (len(shape) - 1))


def _mid(shape):
  return pl.BlockSpec(shape, lambda i: (0, i) + (0,) * (len(shape) - 2))


_GRID = N // BN
_ARB = pltpu.CompilerParams(dimension_semantics=("arbitrary",))

_tc0 = pl.pallas_call(
    _tc0_body,
    grid=(_GRID,),
    in_specs=[_rows((BN, NA)), _full((H, NA)), _full((H, H))],
    out_specs=[_rows((BN, H)), _rows((BN, H))],
    out_shape=[jax.ShapeDtypeStruct((N, H), jnp.float32),
               jax.ShapeDtypeStruct((N, H), jnp.float32)],
    compiler_params=_ARB,
)

_GRU_SPECS = [
    _rows((BN, H)),
    _mid((NC, BN, H)),
    _mid((NC, BN, EW)),
    _full((H, H)),
    _full((H, EW)),
    _full((3 * H, H)),
    _full((3 * H, H)),
    _full((1, 3 * H)),
    _full((1, 3 * H)),
]

_tcr = pl.pallas_call(
    _tcr_body,
    grid=(_GRID,),
    in_specs=_GRU_SPECS + [_full((H, H))],
    out_specs=[_rows((BN, H)), _rows((BN, H))],
    out_shape=[jax.ShapeDtypeStruct((N, H), jnp.float32),
               jax.ShapeDtypeStruct((N, H), jnp.float32)],
    compiler_params=_ARB,
)

_tc3 = pl.pallas_call(
    _tc3_body,
    grid=(_GRID,),
    in_specs=_GRU_SPECS + [
        _full((2 * H, H)), _full((1, 2 * H)),
        _full((2 * H, H)), _full((1, 2 * H)),
        _full((H, H)), _full((1, H)),
        _full((H, H)), _full((1, H)),
        _full((H, H)), _full((1, H)),
        _full((H, H)), _full((1, H)),
    ],
    out_specs=[_full((1, 4 * H))],
    out_shape=[jax.ShapeDtypeStruct((1, 4 * H), jnp.float32)],
    scratch_shapes=[pltpu.VMEM((1, 2 * H), jnp.float32),
                    pltpu.VMEM((1, H), jnp.float32)],
    compiler_params=_ARB,
)


@jax.jit
def kernel(x, edge_attr, condition, node_emb_W, edge_emb_W, U_W, U_b,
           gru_Wih, gru_Whh, gru_bih, gru_bhh, gv_W, gv_b, gvc_W, gvc_b,
           ev_W, ev_b, evc_W, evc_b, mean_W, mean_b, logvar_W, logvar_b,
           edge_index):
  src = edge_index[0].astype(jnp.int32).reshape(NW, NCHUNK, CHUNK)
  dst = edge_index[1].astype(jnp.int32).reshape(NW, NCHUNK, CHUNK)
  epad = jnp.concatenate(
      [edge_attr.astype(jnp.float32),
       jnp.ones((E, 1), jnp.float32),
       jnp.zeros((E, EW - DEG_COL - 1), jnp.float32)],
      axis=1).reshape(NW, NCHUNK, CHUNK, EW)

  Ud = U_W[:, :, :H]
  Us = U_W[:, :, H:2 * H]
  Ue = U_W[:, :, 2 * H:2 * H + edge_emb_W.shape[0]]
  Uc = U_W[:, :, 2 * H + edge_emb_W.shape[0]:]
  cvec = jnp.einsum("rhc,c->rh", Uc, condition) + U_b
  W16 = jnp.concatenate(
      [jnp.einsum("rhe,eb->rhb", Ue, edge_emb_W),
       cvec[:, :, None],
       jnp.zeros((3, H, EW - DEG_COL - 1), jnp.float32)], axis=2)

  zeros128 = jnp.zeros((N, H), jnp.float32)
  zeros16 = jnp.zeros((N, EW), jnp.float32)
  row = lambda b: b.reshape(1, -1)

  h, b = _tc0(x, node_emb_W, Us[0])
  p, es = _sc_scatter_pre(b, src, dst, zeros128, zeros16, epad)
  out = None
  for i in range(3):
    gru_args = (p, es, Ud[i], W16[i], gru_Wih[i], gru_Whh[i],
                row(gru_bih[i]), row(gru_bhh[i]))
    if i < 2:
      h, b = _tcr(h, *gru_args, Us[i + 1])
      (p,) = _sc_scatter(b, src, dst, zeros128)
    else:
      (out,) = _tc3(h, *gru_args, gv_W, row(gv_b), gvc_W, row(gvc_b),
                    ev_W, row(ev_b), evc_W, row(evc_b),
                    mean_W, row(mean_b), logvar_W, row(logvar_b))
  return out.reshape(4 * H)


# speedup vs baseline: 7.0026x; 7.0026x over previous
import functools

import jax
import jax.numpy as jnp
from jax import lax
from jax.experimental import pallas as pl
from jax.experimental.pallas import tpu as pltpu
from jax.experimental.pallas import tpu_sc as plsc

N = 10000
E = 320000
H = 128
NA = 19
EW = 16
DEG_COL = 10

NC = 2
NS = 16
NW = NC * NS
EPW = E // NW
CHUNK = 80
NCHUNK = EPW // CHUNK
RPT0 = 624
RPT1 = N - (NS - 1) * RPT0

BN = 1000


def _sc_mesh():
  return plsc.VectorSubcoreMesh(
      core_axis_name="c", subcore_axis_name="s",
      num_cores=NC, num_subcores=NS)


def _tile_helpers():
  c = lax.axis_index("c")
  s = lax.axis_index("s")
  w = s * NC + c

  def per_tile_rows(copy_fn):
    @pl.when(s < NS - 1)
    def _():
      copy_fn(pl.multiple_of(s * RPT0, RPT0), RPT0)

    @pl.when(s == NS - 1)
    def _():
      copy_fn((NS - 1) * RPT0, RPT1)

  return c, s, w, per_tile_rows


def _make_sc_scatter():
  def body(b_hbm, srcr, dstr, zeros_hbm,
           p_out, src_v, dst_v, rows_v, p_sh, sem):
    c, s, w, per_tile_rows = _tile_helpers()

    pltpu.sync_copy(srcr.at[w], src_v)
    pltpu.sync_copy(dstr.at[w], dst_v)
    per_tile_rows(lambda r0, nr: pltpu.sync_copy(
        zeros_hbm.at[pl.ds(r0, nr)], p_sh.at[pl.ds(r0, nr)]))
    plsc.subcore_barrier()

    @pl.loop(0, NCHUNK)
    def _(i):
      pltpu.async_copy(b_hbm.at[src_v.at[i]], rows_v, sem).wait()
      pltpu.sync_copy(rows_v, p_sh.at[dst_v.at[i]], add=True)

    plsc.subcore_barrier()
    per_tile_rows(lambda r0, nr: pltpu.sync_copy(
        p_sh.at[pl.ds(r0, nr)], p_out.at[c, pl.ds(r0, nr)]))

  return pl.kernel(
      body,
      out_type=[jax.ShapeDtypeStruct((NC, N, H), jnp.float32)],
      mesh=_sc_mesh(),
      scratch_types=[
          pltpu.VMEM((NCHUNK, CHUNK), jnp.int32),
          pltpu.VMEM((NCHUNK, CHUNK), jnp.int32),
          pltpu.VMEM((CHUNK, H), jnp.float32),
          pltpu.VMEM_SHARED((N, H), jnp.float32),
          pltpu.SemaphoreType.DMA,
      ],
      name="sc_scatter")


def _make_sc_pre():
  def body(dstr, zeros16_hbm, epad_hbm, e_out, dst_v, ebuf_v, e_sh):
    c, s, w, per_tile_rows = _tile_helpers()

    pltpu.sync_copy(dstr.at[w], dst_v)
    per_tile_rows(lambda r0, nr: pltpu.sync_copy(
        zeros16_hbm.at[pl.ds(r0, nr)], e_sh.at[pl.ds(r0, nr)]))
    plsc.subcore_barrier()

    @pl.loop(0, NCHUNK)
    def _(i):
      pltpu.sync_copy(epad_hbm.at[w, i], ebuf_v)
      pltpu.sync_copy(ebuf_v, e_sh.at[dst_v.at[i]], add=True)

    plsc.subcore_barrier()
    per_tile_rows(lambda r0, nr: pltpu.sync_copy(
        e_sh.at[pl.ds(r0, nr)], e_out.at[c, pl.ds(r0, nr)]))

  return pl.kernel(
      body,
      out_type=[jax.ShapeDtypeStruct((NC, N, EW), jnp.float32)],
      mesh=_sc_mesh(),
      scratch_types=[
          pltpu.VMEM((NCHUNK, CHUNK), jnp.int32),
          pltpu.VMEM((CHUNK, EW), jnp.float32),
          pltpu.VMEM_SHARED((N, EW), jnp.float32),
      ],
      compiler_params=pltpu.CompilerParams(use_tc_tiling_on_sc=False),
      name="sc_pre")


_make_sc_scatter = functools.cache(_make_sc_scatter)
_make_sc_pre = functools.cache(_make_sc_pre)


def _dot_t(a, w):
  return lax.dot_general(a, w, (((1,), (1,)), ((), ())),
                         preferred_element_type=jnp.float32)


def _tc0_body(x_ref, wn_ref, us_ref, h_ref, b_ref):
  h = _dot_t(x_ref[...], wn_ref[...])
  h_ref[...] = h
  b_ref[...] = _dot_t(h, us_ref[...])


def _gru_block(h, p_ref, es_ref, ud_ref, w16_ref, wih_ref, whh_ref,
               bih_ref, bhh_ref):
  es = es_ref[0] + es_ref[1]
  deg = es[:, DEG_COL:DEG_COL + 1]
  init = deg * _dot_t(h, ud_ref[...]) + _dot_t(es, w16_ref[...])
  agg = init + p_ref[0] + p_ref[1]
  gi = _dot_t(agg, wih_ref[...]) + bih_ref[...]
  gh = _dot_t(h, whh_ref[...]) + bhh_ref[...]
  r = jax.nn.sigmoid(gi[:, :H] + gh[:, :H])
  z = jax.nn.sigmoid(gi[:, H:2 * H] + gh[:, H:2 * H])
  n = jnp.tanh(gi[:, 2 * H:] + r * gh[:, 2 * H:])
  return (1.0 - z) * n + z * h


def _tcr_body(h_ref, p_ref, es_ref, ud_ref, w16_ref, wih_ref, whh_ref,
              bih_ref, bhh_ref, usn_ref, h_out_ref, b_out_ref):
  hn = _gru_block(h_ref[...], p_ref, es_ref, ud_ref, w16_ref,
                  wih_ref, whh_ref, bih_ref, bhh_ref)
  h_out_ref[...] = hn
  b_out_ref[...] = _dot_t(hn, usn_ref[...])


def _tc3_body(h_ref, p_ref, es_ref, ud_ref, w16_ref, wih_ref, whh_ref,
              bih_ref, bhh_ref, gv_w, gv_b, gvc_w, gvc_b, ev_w, ev_b,
              evc_w, evc_b, mean_w, mean_b, logvar_w, logvar_b,
              out_ref, acc_gv, acc_enc):
  hn = _gru_block(h_ref[...], p_ref, es_ref, ud_ref, w16_ref,
                  wih_ref, whh_ref, bih_ref, bhh_ref)
  gvp = jnp.sum(jax.nn.sigmoid(_dot_t(hn, gvc_w[...]) + gvc_b[...])
                * (_dot_t(hn, gv_w[...]) + gv_b[...]), axis=0, keepdims=True)
  encp = jnp.sum(jax.nn.sigmoid(_dot_t(hn, evc_w[...]) + evc_b[...])
                 * (_dot_t(hn, ev_w[...]) + ev_b[...]), axis=0, keepdims=True)
  k = pl.program_id(0)

  @pl.when(k == 0)
  def _():
    acc_gv[...] = gvp
    acc_enc[...] = encp

  @pl.when(k > 0)
  def _():
    acc_gv[...] = acc_gv[...] + gvp
    acc_enc[...] = acc_enc[...] + encp

  @pl.when(k == pl.num_programs(0) - 1)
  def _():
    enc = acc_enc[...]
    mean = _dot_t(enc, mean_w[...]) + mean_b[...]
    logvar = _dot_t(enc, logvar_w[...]) + logvar_b[...]
    out_ref[...] = jnp.concatenate([mean, logvar, acc_gv[...]], axis=1)


def _full(shape):
  return pl.BlockSpec(shape, lambda i: (0,) * len(shape))


def _rows(shape):
  return pl.BlockSpec(shape, lambda i: (i,) + (0,) * (len(shape) - 1))


def _mid(shape):
  return pl.BlockSpec(shape, lambda i: (0, i) + (0,) * (len(shape) - 2))


_GRID = N // BN
_ARB = pltpu.CompilerParams(dimension_semantics=("arbitrary",))

_tc0 = pl.pallas_call(
    _tc0_body,
    grid=(_GRID,),
    in_specs=[_rows((BN, NA)), _full((H, NA)), _full((H, H))],
    out_specs=[_rows((BN, H)), _rows((BN, H))],
    out_shape=[jax.ShapeDtypeStruct((N, H), jnp.float32),
               jax.ShapeDtypeStruct((N, H), jnp.float32)],
    compiler_params=_ARB,
)

_GRU_SPECS = [
    _rows((BN, H)),
    _mid((NC, BN, H)),
    _mid((NC, BN, EW)),
    _full((H, H)),
    _full((H, EW)),
    _full((3 * H, H)),
    _full((3 * H, H)),
    _full((1, 3 * H)),
    _full((1, 3 * H)),
]

_tcr = pl.pallas_call(
    _tcr_body,
    grid=(_GRID,),
    in_specs=_GRU_SPECS + [_full((H, H))],
    out_specs=[_rows((BN, H)), _rows((BN, H))],
    out_shape=[jax.ShapeDtypeStruct((N, H), jnp.float32),
               jax.ShapeDtypeStruct((N, H), jnp.float32)],
    compiler_params=_ARB,
)

_tc3 = pl.pallas_call(
    _tc3_body,
    grid=(_GRID,),
    in_specs=_GRU_SPECS + [
        _full((2 * H, H)), _full((1, 2 * H)),
        _full((2 * H, H)), _full((1, 2 * H)),
        _full((H, H)), _full((1, H)),
        _full((H, H)), _full((1, H)),
        _full((H, H)), _full((1, H)),
        _full((H, H)), _full((1, H)),
    ],
    out_specs=[_full((1, 4 * H))],
    out_shape=[jax.ShapeDtypeStruct((1, 4 * H), jnp.float32)],
    scratch_shapes=[pltpu.VMEM((1, 2 * H), jnp.float32),
                    pltpu.VMEM((1, H), jnp.float32)],
    compiler_params=_ARB,
)


@jax.jit
def kernel(x, edge_attr, condition, node_emb_W, edge_emb_W, U_W, U_b,
           gru_Wih, gru_Whh, gru_bih, gru_bhh, gv_W, gv_b, gvc_W, gvc_b,
           ev_W, ev_b, evc_W, evc_b, mean_W, mean_b, logvar_W, logvar_b,
           edge_index):
  src = edge_index[0].astype(jnp.int32).reshape(NW, NCHUNK, CHUNK)
  dst = edge_index[1].astype(jnp.int32).reshape(NW, NCHUNK, CHUNK)
  epad = jnp.concatenate(
      [edge_attr.astype(jnp.float32),
       jnp.ones((E, 1), jnp.float32),
       jnp.zeros((E, EW - DEG_COL - 1), jnp.float32)],
      axis=1).reshape(NW, NCHUNK, CHUNK, EW)

  Ud = U_W[:, :, :H]
  Us = U_W[:, :, H:2 * H]
  Ue = U_W[:, :, 2 * H:2 * H + edge_emb_W.shape[0]]
  Uc = U_W[:, :, 2 * H + edge_emb_W.shape[0]:]
  cvec = jnp.einsum("rhc,c->rh", Uc, condition) + U_b
  W16 = jnp.concatenate(
      [jnp.einsum("rhe,eb->rhb", Ue, edge_emb_W),
       cvec[:, :, None],
       jnp.zeros((3, H, EW - DEG_COL - 1), jnp.float32)], axis=2)

  zeros128 = jnp.zeros((N, H), jnp.float32)
  zeros16 = jnp.zeros((N, EW), jnp.float32)
  row = lambda b: b.reshape(1, -1)

  h, b = _tc0(x, node_emb_W, Us[0])
  (es,) = _make_sc_pre()(dst, zeros16, epad)
  (p,) = _make_sc_scatter()(b, src, dst, zeros128)
  out = None
  for i in range(3):
    gru_args = (p, es, Ud[i], W16[i], gru_Wih[i], gru_Whh[i],
                row(gru_bih[i]), row(gru_bhh[i]))
    if i < 2:
      h, b = _tcr(h, *gru_args, Us[i + 1])
      (p,) = _make_sc_scatter()(b, src, dst, zeros128)
    else:
      (out,) = _tc3(h, *gru_args, gv_W, row(gv_b), gvc_W, row(gvc_b),
                    ev_W, row(ev_b), evc_W, row(evc_b),
                    mean_W, row(mean_b), logvar_W, row(logvar_b))
  return out.reshape(4 * H)


# speedup vs baseline: 9.0650x; 1.2945x over previous
import functools

import jax
import jax.numpy as jnp
from jax import lax
from jax.experimental import pallas as pl
from jax.experimental.pallas import tpu as pltpu
from jax.experimental.pallas import tpu_sc as plsc

N = 10000
E = 320000
H = 128
NA = 19
EW = 16
DEG_COL = 10

NC = 2
NS = 16
NW = NC * NS
EPW = E // NW
CHUNK = 40
NCHUNK = EPW // CHUNK
NB = 5
RPT0 = 624
RPT1 = N - (NS - 1) * RPT0

BN = 1000


def _sc_mesh():
  return plsc.VectorSubcoreMesh(
      core_axis_name="c", subcore_axis_name="s",
      num_cores=NC, num_subcores=NS)


def _tile_helpers():
  c = lax.axis_index("c")
  s = lax.axis_index("s")
  w = s * NC + c

  def per_tile_rows(copy_fn):
    @pl.when(s < NS - 1)
    def _():
      copy_fn(pl.multiple_of(s * RPT0, RPT0), RPT0)

    @pl.when(s == NS - 1)
    def _():
      copy_fn((NS - 1) * RPT0, RPT1)

  return c, s, w, per_tile_rows


def _make_sc_scatter():
  def body(b_hbm, srcr, dstr, zeros_hbm,
           p_out, src_v, dst_v, rows_v, p_sh, gsem, ssem):
    c, s, w, per_tile_rows = _tile_helpers()

    def gather_desc(ci, b):
      return pltpu.make_async_copy(
          b_hbm.at[src_v.at[ci]], rows_v.at[b], gsem.at[b])

    def scatter_desc(ci, b):
      return pltpu.make_async_copy(
          rows_v.at[b], p_sh.at[dst_v.at[ci]], ssem.at[b])

    pltpu.sync_copy(srcr.at[w], src_v)
    pltpu.sync_copy(dstr.at[w], dst_v)
    per_tile_rows(lambda r0, nr: pltpu.sync_copy(
        zeros_hbm.at[pl.ds(r0, nr)], p_sh.at[pl.ds(r0, nr)]))
    plsc.subcore_barrier()

    @pl.loop(0, NCHUNK // NB)
    def _(g):
      for b in range(NB):
        ci = g * NB + b

        @pl.when(g > 0)
        def _():
          scatter_desc(ci - NB, b).wait()

        pltpu.async_copy(b_hbm.at[src_v.at[ci]], rows_v.at[b], gsem.at[b])
      for b in range(NB):
        ci = g * NB + b
        gather_desc(ci, b).wait()
        pltpu.async_copy(rows_v.at[b], p_sh.at[dst_v.at[ci]], ssem.at[b],
                         add=True)

    for b in range(NB):
      scatter_desc(NCHUNK - NB + b, b).wait()

    plsc.subcore_barrier()
    per_tile_rows(lambda r0, nr: pltpu.sync_copy(
        p_sh.at[pl.ds(r0, nr)], p_out.at[c, pl.ds(r0, nr)]))

  return pl.kernel(
      body,
      out_type=[jax.ShapeDtypeStruct((NC, N, H), jnp.float32)],
      mesh=_sc_mesh(),
      scratch_types=[
          pltpu.VMEM((NCHUNK, CHUNK), jnp.int32),
          pltpu.VMEM((NCHUNK, CHUNK), jnp.int32),
          pltpu.VMEM((NB, CHUNK, H), jnp.float32),
          pltpu.VMEM_SHARED((N, H), jnp.float32),
          pltpu.SemaphoreType.DMA((NB,)),
          pltpu.SemaphoreType.DMA((NB,)),
      ],
      compiler_params=pltpu.CompilerParams(use_tc_tiling_on_sc=False),
      name="sc_scatter")


def _make_sc_pre():
  def body(dstr, zeros16_hbm, epad_hbm, e_out, dst_v, ebuf_v, e_sh):
    c, s, w, per_tile_rows = _tile_helpers()

    pltpu.sync_copy(dstr.at[w], dst_v)
    per_tile_rows(lambda r0, nr: pltpu.sync_copy(
        zeros16_hbm.at[pl.ds(r0, nr)], e_sh.at[pl.ds(r0, nr)]))
    plsc.subcore_barrier()

    @pl.loop(0, NCHUNK)
    def _(i):
      pltpu.sync_copy(epad_hbm.at[w, i], ebuf_v)
      pltpu.sync_copy(ebuf_v, e_sh.at[dst_v.at[i]], add=True)

    plsc.subcore_barrier()
    per_tile_rows(lambda r0, nr: pltpu.sync_copy(
        e_sh.at[pl.ds(r0, nr)], e_out.at[c, pl.ds(r0, nr)]))

  return pl.kernel(
      body,
      out_type=[jax.ShapeDtypeStruct((NC, N, EW), jnp.float32)],
      mesh=_sc_mesh(),
      scratch_types=[
          pltpu.VMEM((NCHUNK, CHUNK), jnp.int32),
          pltpu.VMEM((CHUNK, EW), jnp.float32),
          pltpu.VMEM_SHARED((N, EW), jnp.float32),
      ],
      compiler_params=pltpu.CompilerParams(use_tc_tiling_on_sc=False),
      name="sc_pre")


_make_sc_scatter = functools.cache(_make_sc_scatter)
_make_sc_pre = functools.cache(_make_sc_pre)


def _dot_t(a, w):
  return lax.dot_general(a, w, (((1,), (1,)), ((), ())),
                         preferred_element_type=jnp.float32)


def _tc0_body(x_ref, wn_ref, us_ref, h_ref, b_ref):
  h = _dot_t(x_ref[...], wn_ref[...])
  h_ref[...] = h
  b_ref[...] = _dot_t(h, us_ref[...])


def _gru_block(h, p_ref, es_ref, ud_ref, w16_ref, wih_ref, whh_ref,
               bih_ref, bhh_ref):
  es = es_ref[0] + es_ref[1]
  deg = es[:, DEG_COL:DEG_COL + 1]
  init = deg * _dot_t(h, ud_ref[...]) + _dot_t(es, w16_ref[...])
  agg = init + p_ref[0] + p_ref[1]
  gi = _dot_t(agg, wih_ref[...]) + bih_ref[...]
  gh = _dot_t(h, whh_ref[...]) + bhh_ref[...]
  r = jax.nn.sigmoid(gi[:, :H] + gh[:, :H])
  z = jax.nn.sigmoid(gi[:, H:2 * H] + gh[:, H:2 * H])
  n = jnp.tanh(gi[:, 2 * H:] + r * gh[:, 2 * H:])
  return (1.0 - z) * n + z * h


def _tcr_body(h_ref, p_ref, es_ref, ud_ref, w16_ref, wih_ref, whh_ref,
              bih_ref, bhh_ref, usn_ref, h_out_ref, b_out_ref):
  hn = _gru_block(h_ref[...], p_ref, es_ref, ud_ref, w16_ref,
                  wih_ref, whh_ref, bih_ref, bhh_ref)
  h_out_ref[...] = hn
  b_out_ref[...] = _dot_t(hn, usn_ref[...])


def _tc3_body(h_ref, p_ref, es_ref, ud_ref, w16_ref, wih_ref, whh_ref,
              bih_ref, bhh_ref, gv_w, gv_b, gvc_w, gvc_b, ev_w, ev_b,
              evc_w, evc_b, mean_w, mean_b, logvar_w, logvar_b,
              out_ref, acc_gv, acc_enc):
  hn = _gru_block(h_ref[...], p_ref, es_ref, ud_ref, w16_ref,
                  wih_ref, whh_ref, bih_ref, bhh_ref)
  gvp = jnp.sum(jax.nn.sigmoid(_dot_t(hn, gvc_w[...]) + gvc_b[...])
                * (_dot_t(hn, gv_w[...]) + gv_b[...]), axis=0, keepdims=True)
  encp = jnp.sum(jax.nn.sigmoid(_dot_t(hn, evc_w[...]) + evc_b[...])
                 * (_dot_t(hn, ev_w[...]) + ev_b[...]), axis=0, keepdims=True)
  k = pl.program_id(0)

  @pl.when(k == 0)
  def _():
    acc_gv[...] = gvp
    acc_enc[...] = encp

  @pl.when(k > 0)
  def _():
    acc_gv[...] = acc_gv[...] + gvp
    acc_enc[...] = acc_enc[...] + encp

  @pl.when(k == pl.num_programs(0) - 1)
  def _():
    enc = acc_enc[...]
    mean = _dot_t(enc, mean_w[...]) + mean_b[...]
    logvar = _dot_t(enc, logvar_w[...]) + logvar_b[...]
    out_ref[...] = jnp.concatenate([mean, logvar, acc_gv[...]], axis=1)


def _full(shape):
  return pl.BlockSpec(shape, lambda i: (0,) * len(shape))


def _rows(shape):
  return pl.BlockSpec(shape, lambda i: (i,) + (0,) * (len(shape) - 1))


def _mid(shape):
  return pl.BlockSpec(shape, lambda i: (0, i) + (0,) * (len(shape) - 2))


_GRID = N // BN
_ARB = pltpu.CompilerParams(dimension_semantics=("arbitrary",))

_tc0 = pl.pallas_call(
    _tc0_body,
    grid=(_GRID,),
    in_specs=[_rows((BN, NA)), _full((H, NA)), _full((H, H))],
    out_specs=[_rows((BN, H)), _rows((BN, H))],
    out_shape=[jax.ShapeDtypeStruct((N, H), jnp.float32),
               jax.ShapeDtypeStruct((N, H), jnp.float32)],
    compiler_params=_ARB,
)

_GRU_SPECS = [
    _rows((BN, H)),
    _mid((NC, BN, H)),
    _mid((NC, BN, EW)),
    _full((H, H)),
    _full((H, EW)),
    _full((3 * H, H)),
    _full((3 * H, H)),
    _full((1, 3 * H)),
    _full((1, 3 * H)),
]

_tcr = pl.pallas_call(
    _tcr_body,
    grid=(_GRID,),
    in_specs=_GRU_SPECS + [_full((H, H))],
    out_specs=[_rows((BN, H)), _rows((BN, H))],
    out_shape=[jax.ShapeDtypeStruct((N, H), jnp.float32),
               jax.ShapeDtypeStruct((N, H), jnp.float32)],
    compiler_params=_ARB,
)

_tc3 = pl.pallas_call(
    _tc3_body,
    grid=(_GRID,),
    in_specs=_GRU_SPECS + [
        _full((2 * H, H)), _full((1, 2 * H)),
        _full((2 * H, H)), _full((1, 2 * H)),
        _full((H, H)), _full((1, H)),
        _full((H, H)), _full((1, H)),
        _full((H, H)), _full((1, H)),
        _full((H, H)), _full((1, H)),
    ],
    out_specs=[_full((1, 4 * H))],
    out_shape=[jax.ShapeDtypeStruct((1, 4 * H), jnp.float32)],
    scratch_shapes=[pltpu.VMEM((1, 2 * H), jnp.float32),
                    pltpu.VMEM((1, H), jnp.float32)],
    compiler_params=_ARB,
)


@jax.jit
def kernel(x, edge_attr, condition, node_emb_W, edge_emb_W, U_W, U_b,
           gru_Wih, gru_Whh, gru_bih, gru_bhh, gv_W, gv_b, gvc_W, gvc_b,
           ev_W, ev_b, evc_W, evc_b, mean_W, mean_b, logvar_W, logvar_b,
           edge_index):
  src = edge_index[0].astype(jnp.int32).reshape(NW, NCHUNK, CHUNK)
  dst = edge_index[1].astype(jnp.int32).reshape(NW, NCHUNK, CHUNK)
  epad = jnp.concatenate(
      [edge_attr.astype(jnp.float32),
       jnp.ones((E, 1), jnp.float32),
       jnp.zeros((E, EW - DEG_COL - 1), jnp.float32)],
      axis=1).reshape(NW, NCHUNK, CHUNK, EW)

  Ud = U_W[:, :, :H]
  Us = U_W[:, :, H:2 * H]
  Ue = U_W[:, :, 2 * H:2 * H + edge_emb_W.shape[0]]
  Uc = U_W[:, :, 2 * H + edge_emb_W.shape[0]:]
  cvec = jnp.einsum("rhc,c->rh", Uc, condition) + U_b
  W16 = jnp.concatenate(
      [jnp.einsum("rhe,eb->rhb", Ue, edge_emb_W),
       cvec[:, :, None],
       jnp.zeros((3, H, EW - DEG_COL - 1), jnp.float32)], axis=2)

  zeros128 = jnp.zeros((N, H), jnp.float32)
  zeros16 = jnp.zeros((N, EW), jnp.float32)
  row = lambda b: b.reshape(1, -1)

  h, b = _tc0(x, node_emb_W, Us[0])
  (es,) = _make_sc_pre()(dst, zeros16, epad)
  (p,) = _make_sc_scatter()(b, src, dst, zeros128)
  out = None
  for i in range(3):
    gru_args = (p, es, Ud[i], W16[i], gru_Wih[i], gru_Whh[i],
                row(gru_bih[i]), row(gru_bhh[i]))
    if i < 2:
      h, b = _tcr(h, *gru_args, Us[i + 1])
      (p,) = _make_sc_scatter()(b, src, dst, zeros128)
    else:
      (out,) = _tc3(h, *gru_args, gv_W, row(gv_b), gvc_W, row(gvc_b),
                    ev_W, row(ev_b), evc_W, row(evc_b),
                    mean_W, row(mean_b), logvar_W, row(logvar_b))
  return out.reshape(4 * H)


# speedup vs baseline: 11.1293x; 1.2277x over previous
import functools

import jax
import jax.numpy as jnp
from jax import lax
from jax.experimental import pallas as pl
from jax.experimental.pallas import tpu as pltpu
from jax.experimental.pallas import tpu_sc as plsc

N = 10000
E = 320000
H = 128
NA = 19
EW = 16
DEG_COL = 10

NC = 2
NS = 16
NW = NC * NS
EPW = E // NW
CHUNK = 40
NCHUNK = EPW // CHUNK
NB = 5
CHUNKP = 80
NCHUNKP = EPW // CHUNKP
RPT0 = 624
RPT1 = N - (NS - 1) * RPT0

BN = 1000


def _sc_mesh():
  return plsc.VectorSubcoreMesh(
      core_axis_name="c", subcore_axis_name="s",
      num_cores=NC, num_subcores=NS)


def _tile_helpers():
  c = lax.axis_index("c")
  s = lax.axis_index("s")
  w = s * NC + c

  def per_tile_rows(copy_fn):
    @pl.when(s < NS - 1)
    def _():
      copy_fn(pl.multiple_of(s * RPT0, RPT0), RPT0)

    @pl.when(s == NS - 1)
    def _():
      copy_fn((NS - 1) * RPT0, RPT1)

  return c, s, w, per_tile_rows


def _make_sc_scatter():
  def body(b_hbm, srcr, dstr, zeros_hbm,
           p_out, src_v, dst_v, rows_v, p_sh, gsem, ssem):
    c, s, w, per_tile_rows = _tile_helpers()

    def gather_desc(ci, b):
      return pltpu.make_async_copy(
          b_hbm.at[src_v.at[ci]], rows_v.at[b], gsem.at[b])

    def scatter_desc(ci, b):
      return pltpu.make_async_copy(
          rows_v.at[b], p_sh.at[dst_v.at[ci]], ssem.at[b])

    pltpu.sync_copy(srcr.at[w], src_v)
    pltpu.sync_copy(dstr.at[w], dst_v)
    per_tile_rows(lambda r0, nr: pltpu.sync_copy(
        zeros_hbm.at[pl.ds(r0, nr)], p_sh.at[pl.ds(r0, nr)]))
    plsc.subcore_barrier()

    @pl.loop(0, NCHUNK // NB)
    def _(g):
      for b in range(NB):
        ci = g * NB + b

        @pl.when(g > 0)
        def _():
          scatter_desc(ci - NB, b).wait()

        pltpu.async_copy(b_hbm.at[src_v.at[ci]], rows_v.at[b], gsem.at[b])
      for b in range(NB):
        ci = g * NB + b
        gather_desc(ci, b).wait()
        pltpu.async_copy(rows_v.at[b], p_sh.at[dst_v.at[ci]], ssem.at[b],
                         add=True)

    for b in range(NB):
      scatter_desc(NCHUNK - NB + b, b).wait()

    plsc.subcore_barrier()
    per_tile_rows(lambda r0, nr: pltpu.sync_copy(
        p_sh.at[pl.ds(r0, nr)], p_out.at[c, pl.ds(r0, nr)]))

  return pl.kernel(
      body,
      out_type=[jax.ShapeDtypeStruct((NC, N, H), jnp.float32)],
      mesh=_sc_mesh(),
      scratch_types=[
          pltpu.VMEM((NCHUNK, CHUNK), jnp.int32),
          pltpu.VMEM((NCHUNK, CHUNK), jnp.int32),
          pltpu.VMEM((NB, CHUNK, H), jnp.float32),
          pltpu.VMEM_SHARED((N, H), jnp.float32),
          pltpu.SemaphoreType.DMA((NB,)),
          pltpu.SemaphoreType.DMA((NB,)),
      ],
      compiler_params=pltpu.CompilerParams(use_tc_tiling_on_sc=False),
      name="sc_scatter")


def _make_sc_pre():
  def body(dstr, zeros16_hbm, epad_hbm, e_out, dst_v, ebuf_v, e_sh,
           lsem, ssem):
    c, s, w, per_tile_rows = _tile_helpers()

    def load_desc(ci, b):
      return pltpu.make_async_copy(
          epad_hbm.at[w, ci], ebuf_v.at[b], lsem.at[b])

    def scatter_desc(ci, b):
      return pltpu.make_async_copy(
          ebuf_v.at[b], e_sh.at[dst_v.at[ci]], ssem.at[b])

    pltpu.sync_copy(dstr.at[w], dst_v)
    per_tile_rows(lambda r0, nr: pltpu.sync_copy(
        zeros16_hbm.at[pl.ds(r0, nr)], e_sh.at[pl.ds(r0, nr)]))
    plsc.subcore_barrier()

    @pl.loop(0, NCHUNKP // NB)
    def _(g):
      for b in range(NB):
        ci = g * NB + b

        @pl.when(g > 0)
        def _():
          scatter_desc(ci - NB, b).wait()

        pltpu.async_copy(epad_hbm.at[w, ci], ebuf_v.at[b], lsem.at[b])
      for b in range(NB):
        ci = g * NB + b
        load_desc(ci, b).wait()
        pltpu.async_copy(ebuf_v.at[b], e_sh.at[dst_v.at[ci]], ssem.at[b],
                         add=True)

    for b in range(NB):
      scatter_desc(NCHUNKP - NB + b, b).wait()

    plsc.subcore_barrier()
    per_tile_rows(lambda r0, nr: pltpu.sync_copy(
        e_sh.at[pl.ds(r0, nr)], e_out.at[c, pl.ds(r0, nr)]))

  return pl.kernel(
      body,
      out_type=[jax.ShapeDtypeStruct((NC, N, EW), jnp.float32)],
      mesh=_sc_mesh(),
      scratch_types=[
          pltpu.VMEM((NCHUNKP, CHUNKP), jnp.int32),
          pltpu.VMEM((NB, CHUNKP, EW), jnp.float32),
          pltpu.VMEM_SHARED((N, EW), jnp.float32),
          pltpu.SemaphoreType.DMA((NB,)),
          pltpu.SemaphoreType.DMA((NB,)),
      ],
      compiler_params=pltpu.CompilerParams(use_tc_tiling_on_sc=False),
      name="sc_pre")


_make_sc_scatter = functools.cache(_make_sc_scatter)
_make_sc_pre = functools.cache(_make_sc_pre)


def _dot_t(a, w):
  return lax.dot_general(a, w, (((1,), (1,)), ((), ())),
                         preferred_element_type=jnp.float32)


def _tc0_body(x_ref, wn_ref, us_ref, h_ref, b_ref):
  h = _dot_t(x_ref[...], wn_ref[...])
  h_ref[...] = h
  b_ref[...] = _dot_t(h, us_ref[...])


def _gru_block(h, p_ref, es_ref, ud_ref, w16_ref, wih_ref, whh_ref,
               bih_ref, bhh_ref):
  es = es_ref[0] + es_ref[1]
  deg = es[:, DEG_COL:DEG_COL + 1]
  init = deg * _dot_t(h, ud_ref[...]) + _dot_t(es, w16_ref[...])
  agg = init + p_ref[0] + p_ref[1]
  gi = _dot_t(agg, wih_ref[...]) + bih_ref[...]
  gh = _dot_t(h, whh_ref[...]) + bhh_ref[...]
  r = jax.nn.sigmoid(gi[:, :H] + gh[:, :H])
  z = jax.nn.sigmoid(gi[:, H:2 * H] + gh[:, H:2 * H])
  n = jnp.tanh(gi[:, 2 * H:] + r * gh[:, 2 * H:])
  return (1.0 - z) * n + z * h


def _tcr_body(h_ref, p_ref, es_ref, ud_ref, w16_ref, wih_ref, whh_ref,
              bih_ref, bhh_ref, usn_ref, h_out_ref, b_out_ref):
  hn = _gru_block(h_ref[...], p_ref, es_ref, ud_ref, w16_ref,
                  wih_ref, whh_ref, bih_ref, bhh_ref)
  h_out_ref[...] = hn
  b_out_ref[...] = _dot_t(hn, usn_ref[...])


def _tc3_body(h_ref, p_ref, es_ref, ud_ref, w16_ref, wih_ref, whh_ref,
              bih_ref, bhh_ref, gv_w, gv_b, gvc_w, gvc_b, ev_w, ev_b,
              evc_w, evc_b, mean_w, mean_b, logvar_w, logvar_b,
              out_ref, acc_gv, acc_enc):
  hn = _gru_block(h_ref[...], p_ref, es_ref, ud_ref, w16_ref,
                  wih_ref, whh_ref, bih_ref, bhh_ref)
  gvp = jnp.sum(jax.nn.sigmoid(_dot_t(hn, gvc_w[...]) + gvc_b[...])
                * (_dot_t(hn, gv_w[...]) + gv_b[...]), axis=0, keepdims=True)
  encp = jnp.sum(jax.nn.sigmoid(_dot_t(hn, evc_w[...]) + evc_b[...])
                 * (_dot_t(hn, ev_w[...]) + ev_b[...]), axis=0, keepdims=True)
  k = pl.program_id(0)

  @pl.when(k == 0)
  def _():
    acc_gv[...] = gvp
    acc_enc[...] = encp

  @pl.when(k > 0)
  def _():
    acc_gv[...] = acc_gv[...] + gvp
    acc_enc[...] = acc_enc[...] + encp

  @pl.when(k == pl.num_programs(0) - 1)
  def _():
    enc = acc_enc[...]
    mean = _dot_t(enc, mean_w[...]) + mean_b[...]
    logvar = _dot_t(enc, logvar_w[...]) + logvar_b[...]
    out_ref[...] = jnp.concatenate([mean, logvar, acc_gv[...]], axis=1)


def _full(shape):
  return pl.BlockSpec(shape, lambda i: (0,) * len(shape))


def _rows(shape):
  return pl.BlockSpec(shape, lambda i: (i,) + (0,) * (len(shape) - 1))


def _mid(shape):
  return pl.BlockSpec(shape, lambda i: (0, i) + (0,) * (len(shape) - 2))


_GRID = N // BN
_ARB = pltpu.CompilerParams(dimension_semantics=("arbitrary",))

_tc0 = pl.pallas_call(
    _tc0_body,
    grid=(_GRID,),
    in_specs=[_rows((BN, NA)), _full((H, NA)), _full((H, H))],
    out_specs=[_rows((BN, H)), _rows((BN, H))],
    out_shape=[jax.ShapeDtypeStruct((N, H), jnp.float32),
               jax.ShapeDtypeStruct((N, H), jnp.float32)],
    compiler_params=_ARB,
)

_GRU_SPECS = [
    _rows((BN, H)),
    _mid((NC, BN, H)),
    _mid((NC, BN, EW)),
    _full((H, H)),
    _full((H, EW)),
    _full((3 * H, H)),
    _full((3 * H, H)),
    _full((1, 3 * H)),
    _full((1, 3 * H)),
]

_tcr = pl.pallas_call(
    _tcr_body,
    grid=(_GRID,),
    in_specs=_GRU_SPECS + [_full((H, H))],
    out_specs=[_rows((BN, H)), _rows((BN, H))],
    out_shape=[jax.ShapeDtypeStruct((N, H), jnp.float32),
               jax.ShapeDtypeStruct((N, H), jnp.float32)],
    compiler_params=_ARB,
)

_tc3 = pl.pallas_call(
    _tc3_body,
    grid=(_GRID,),
    in_specs=_GRU_SPECS + [
        _full((2 * H, H)), _full((1, 2 * H)),
        _full((2 * H, H)), _full((1, 2 * H)),
        _full((H, H)), _full((1, H)),
        _full((H, H)), _full((1, H)),
        _full((H, H)), _full((1, H)),
        _full((H, H)), _full((1, H)),
    ],
    out_specs=[_full((1, 4 * H))],
    out_shape=[jax.ShapeDtypeStruct((1, 4 * H), jnp.float32)],
    scratch_shapes=[pltpu.VMEM((1, 2 * H), jnp.float32),
                    pltpu.VMEM((1, H), jnp.float32)],
    compiler_params=_ARB,
)


@jax.jit
def kernel(x, edge_attr, condition, node_emb_W, edge_emb_W, U_W, U_b,
           gru_Wih, gru_Whh, gru_bih, gru_bhh, gv_W, gv_b, gvc_W, gvc_b,
           ev_W, ev_b, evc_W, evc_b, mean_W, mean_b, logvar_W, logvar_b,
           edge_index):
  src = edge_index[0].astype(jnp.int32).reshape(NW, NCHUNK, CHUNK)
  dst = edge_index[1].astype(jnp.int32).reshape(NW, NCHUNK, CHUNK)
  dstp = edge_index[1].astype(jnp.int32).reshape(NW, NCHUNKP, CHUNKP)
  epad = jnp.concatenate(
      [edge_attr.astype(jnp.float32),
       jnp.ones((E, 1), jnp.float32),
       jnp.zeros((E, EW - DEG_COL - 1), jnp.float32)],
      axis=1).reshape(NW, NCHUNKP, CHUNKP, EW)

  Ud = U_W[:, :, :H]
  Us = U_W[:, :, H:2 * H]
  Ue = U_W[:, :, 2 * H:2 * H + edge_emb_W.shape[0]]
  Uc = U_W[:, :, 2 * H + edge_emb_W.shape[0]:]
  cvec = jnp.einsum("rhc,c->rh", Uc, condition) + U_b
  W16 = jnp.concatenate(
      [jnp.einsum("rhe,eb->rhb", Ue, edge_emb_W),
       cvec[:, :, None],
       jnp.zeros((3, H, EW - DEG_COL - 1), jnp.float32)], axis=2)

  zeros128 = jnp.zeros((N, H), jnp.float32)
  zeros16 = jnp.zeros((N, EW), jnp.float32)
  row = lambda b: b.reshape(1, -1)

  h, b = _tc0(x, node_emb_W, Us[0])
  (es,) = _make_sc_pre()(dstp, zeros16, epad)
  (p,) = _make_sc_scatter()(b, src, dst, zeros128)
  out = None
  for i in range(3):
    gru_args = (p, es, Ud[i], W16[i], gru_Wih[i], gru_Whh[i],
                row(gru_bih[i]), row(gru_bhh[i]))
    if i < 2:
      h, b = _tcr(h, *gru_args, Us[i + 1])
      (p,) = _make_sc_scatter()(b, src, dst, zeros128)
    else:
      (out,) = _tc3(h, *gru_args, gv_W, row(gv_b), gvc_W, row(gvc_b),
                    ev_W, row(ev_b), evc_W, row(evc_b),
                    mean_W, row(mean_b), logvar_W, row(logvar_b))
  return out.reshape(4 * H)
